# baseline (speedup 1.0000x reference)
import functools

import jax
import jax.numpy as jnp
from jax import lax
from jax.experimental import pallas as pl
from jax.experimental.pallas import tpu as pltpu
from jax.experimental.pallas import tpu_sc as plsc

N_NODES = 10000
D = 128
N_EDGES = 320000

NC = 2
NS = 16
NW = NC * NS
CH = 128
L = 16

NPAD = 10240
ROWS_PER_TILE = NPAD // NS
ZCH = 128

NBUF = 2
EPW_CH = NBUF * (-(-N_EDGES // (NW * CH * NBUF)))
EPW = EPW_CH * CH
EPAD = EPW * NW


def _sc_body(eidx_hbm, x_hbm, zrow_hbm, iota_hbm, acc_out, cnt_out,
             srcv, dstv, idxb, rows, cntloc, acc_sh,
             sem, semi, semg, sems):
    cid = lax.axis_index("c")
    sid = lax.axis_index("s")
    wid = sid * NC + cid

    zero16 = jnp.zeros((L,), jnp.float32)

    def _zero_cnt(i, _):
        cntloc[pl.ds(i * L, L)] = zero16
        return _

    lax.fori_loop(0, NPAD // L, _zero_cnt, None)

    rbase = sid * ROWS_PER_TILE
    pltpu.sync_copy(zrow_hbm, rows.at[0])
    for t in range(ROWS_PER_TILE // ZCH):
        pltpu.sync_copy(iota_hbm.at[pl.ds(rbase + t * ZCH, ZCH)], dstv.at[0])
        pltpu.sync_copy(rows.at[0], acc_sh.at[dstv.at[0]])

    plsc.subcore_barrier()

    cbase = wid * EPW_CH
    ones16 = jnp.ones((L,), jnp.float32)

    def _group(g, _):
        c0 = pl.multiple_of((cbase + g * NBUF) * 2, 2 * NBUF)
        di = [pltpu.async_copy(eidx_hbm.at[pl.ds(c0 + 2 * b, 2)], idxb.at[b],
                               semi[b]) for b in range(NBUF)]
        dg = []
        for b in range(NBUF):
            di[b].wait()
            dg.append(pltpu.async_copy(x_hbm.at[idxb.at[b, 0]], rows.at[b],
                                       semg[b]))
        ds_ = []
        for b in range(NBUF):
            dg[b].wait()
            ds_.append(pltpu.async_copy(rows.at[b], acc_sh.at[idxb.at[b, 1]],
                                        sems[b], add=True))
            for q in range(CH // L):
                d = idxb[b, 1, pl.ds(q * L, L)]
                plsc.addupdate_scatter(cntloc, [d], ones16)
        for b in range(NBUF):
            ds_[b].wait()
        return _

    lax.fori_loop(0, EPW_CH // NBUF, _group, None)

    plsc.subcore_barrier()

    pltpu.sync_copy(cntloc, cnt_out.at[pl.ds(wid * NPAD, NPAD)])
    obase = cid * NPAD + rbase
    for t in range(ROWS_PER_TILE // ZCH):
        pltpu.sync_copy(iota_hbm.at[pl.ds(rbase + t * ZCH, ZCH)], srcv)
        pltpu.async_copy(acc_sh.at[srcv], rows.at[0], sem).wait()
        pltpu.sync_copy(rows.at[0], acc_out.at[pl.ds(obase + t * ZCH, ZCH)])


_sc_aggregate = functools.partial(
    pl.kernel,
    out_type=(
        jax.ShapeDtypeStruct((NC * NPAD, D), jnp.float32),
        jax.ShapeDtypeStruct((NW * NPAD,), jnp.float32),
    ),
    mesh=plsc.VectorSubcoreMesh(core_axis_name="c", subcore_axis_name="s"),
    compiler_params=pltpu.CompilerParams(needs_layout_passes=False),
    scratch_types=[
        pltpu.VMEM((CH,), jnp.int32),
        pltpu.VMEM((1, CH), jnp.int32),
        pltpu.VMEM((NBUF, 2, CH), jnp.int32),
        pltpu.VMEM((NBUF, CH, D), jnp.float32),
        pltpu.VMEM((NPAD,), jnp.float32),
        pltpu.VMEM_SHARED((NPAD, D), jnp.float32),
        pltpu.SemaphoreType.DMA,
        [pltpu.SemaphoreType.DMA] * NBUF,
        [pltpu.SemaphoreType.DMA] * NBUF,
        [pltpu.SemaphoreType.DMA] * NBUF,
    ],
)(_sc_body)


def _tc_body(x_ref, a0_ref, a1_ref, c_ref,
             w1_ref, w2_ref, b_ref, wp_ref, bp_ref, out_ref):
    a = a0_ref[...] + a1_ref[...]
    csum = jnp.sum(c_ref[...], axis=1)
    cnt = jnp.maximum(csum, 1.0)[:, None]
    mean = a / cnt
    h = jnp.dot(x_ref[...], w1_ref[...], preferred_element_type=jnp.float32)
    h = h + jnp.dot(mean, w2_ref[...], preferred_element_type=jnp.float32)
    h = jnp.maximum(h + b_ref[...], 0.0)
    p = jnp.dot(h, wp_ref[...], preferred_element_type=jnp.float32)
    out_ref[...] = jnp.maximum(p + bp_ref[...], 0.0)


def kernel(x, edge_index, W_sage, b_sage, W_proj, b_proj):
    ei = edge_index.astype(jnp.int32)
    npad = EPAD - N_EDGES
    src = jnp.concatenate([ei[0], jnp.zeros((npad,), jnp.int32)])
    dst = jnp.concatenate([ei[1], jnp.full((npad,), N_NODES, jnp.int32)])
    eidx = jnp.stack([src.reshape(-1, CH), dst.reshape(-1, CH)],
                     axis=1).reshape(-1, CH)

    zrow = jnp.zeros((ZCH, D), jnp.float32)
    iota = jnp.arange(NPAD, dtype=jnp.int32)

    acc, cnt = _sc_aggregate(eidx, x, zrow, iota)
    cnt = cnt.reshape(NW, NPAD).T

    R = 1000
    grid = (N_NODES // R,)
    row_spec = pl.BlockSpec((R, D), lambda i: (i, 0))
    cnt_spec = pl.BlockSpec((R, NW), lambda i: (i, 0))

    def w_spec(r, c_):
        return pl.BlockSpec((r, c_), lambda i: (0, 0))

    out = pl.pallas_call(
        _tc_body,
        grid=grid,
        in_specs=[
            row_spec, row_spec, row_spec, cnt_spec,
            w_spec(D, D), w_spec(D, D), w_spec(1, D), w_spec(D, 1), w_spec(1, 1),
        ],
        out_specs=pl.BlockSpec((R, 1), lambda i: (i, 0)),
        out_shape=jax.ShapeDtypeStruct((N_NODES, 1), jnp.float32),
    )(
        x, acc[:N_NODES], acc[NPAD:NPAD + N_NODES], cnt[:N_NODES],
        W_sage[:D], W_sage[D:], b_sage.reshape(1, D),
        W_proj, b_proj.reshape(1, 1),
    )
    return out


# speedup vs baseline: 1.2767x; 1.2767x over previous
import functools

import jax
import jax.numpy as jnp
from jax import lax
from jax.experimental import pallas as pl
from jax.experimental.pallas import tpu as pltpu
from jax.experimental.pallas import tpu_sc as plsc

N_NODES = 10000
D = 128
N_EDGES = 320000

NC = 2
NS = 16
NW = NC * NS
CH = 128
L = 16

NPAD = 10240
ROWS_PER_TILE = NPAD // NS
ZCH = 128

NBUF = 1
EPW_CH = NBUF * (-(-N_EDGES // (NW * CH * NBUF)))
EPW = EPW_CH * CH
EPAD = EPW * NW


def _sc_body(eidx_hbm, x_hbm, zrow_hbm, iota_hbm, acc_out, cnt_out,
             srcv, dstv, idxb, rows, cntloc, acc_sh,
             sem, semi, semg, sems):
    cid = lax.axis_index("c")
    sid = lax.axis_index("s")
    wid = sid * NC + cid

    zero16 = jnp.zeros((L,), jnp.float32)

    def _zero_cnt(i, _):
        cntloc[pl.ds(i * L, L)] = zero16
        return _

    lax.fori_loop(0, NPAD // L, _zero_cnt, None)

    rbase = sid * ROWS_PER_TILE
    pltpu.sync_copy(zrow_hbm, rows.at[0])
    for t in range(ROWS_PER_TILE // ZCH):
        pltpu.sync_copy(iota_hbm.at[pl.ds(rbase + t * ZCH, ZCH)], dstv.at[0])
        pltpu.sync_copy(rows.at[0], acc_sh.at[dstv.at[0]])

    plsc.subcore_barrier()

    cbase = wid * EPW_CH
    ones16 = jnp.ones((L,), jnp.float32)

    def _group(g, _):
        c0 = pl.multiple_of((cbase + g * NBUF) * 2, 2 * NBUF)
        di = [pltpu.async_copy(eidx_hbm.at[pl.ds(c0 + 2 * b, 2)], idxb.at[b],
                               semi[b]) for b in range(NBUF)]
        dg = []
        for b in range(NBUF):
            di[b].wait()
            dg.append(pltpu.async_copy(x_hbm.at[idxb.at[b, 0]], rows.at[b],
                                       semg[b]))
        ds_ = []
        for b in range(NBUF):
            dg[b].wait()
            ds_.append(pltpu.async_copy(rows.at[b], acc_sh.at[idxb.at[b, 1]],
                                        sems[b], add=True))
            for q in range(CH // L):
                d = idxb[b, 1, pl.ds(q * L, L)]
                plsc.addupdate_scatter(cntloc, [d], ones16)
        for b in range(NBUF):
            ds_[b].wait()
        return _

    lax.fori_loop(0, EPW_CH // NBUF, _group, None)

    plsc.subcore_barrier()

    pltpu.sync_copy(cntloc, cnt_out.at[pl.ds(wid * NPAD, NPAD)])
    obase = cid * NPAD + rbase
    for t in range(ROWS_PER_TILE // ZCH):
        pltpu.sync_copy(iota_hbm.at[pl.ds(rbase + t * ZCH, ZCH)], srcv)
        pltpu.async_copy(acc_sh.at[srcv], rows.at[0], sem).wait()
        pltpu.sync_copy(rows.at[0], acc_out.at[pl.ds(obase + t * ZCH, ZCH)])


_sc_aggregate = functools.partial(
    pl.kernel,
    out_type=(
        jax.ShapeDtypeStruct((NC * NPAD, D), jnp.float32),
        jax.ShapeDtypeStruct((NW * NPAD,), jnp.float32),
    ),
    mesh=plsc.VectorSubcoreMesh(core_axis_name="c", subcore_axis_name="s"),
    compiler_params=pltpu.CompilerParams(needs_layout_passes=False),
    scratch_types=[
        pltpu.VMEM((CH,), jnp.int32),
        pltpu.VMEM((1, CH), jnp.int32),
        pltpu.VMEM((NBUF, 2, CH), jnp.int32),
        pltpu.VMEM((NBUF, CH, D), jnp.float32),
        pltpu.VMEM((NPAD,), jnp.float32),
        pltpu.VMEM_SHARED((NPAD, D), jnp.float32),
        pltpu.SemaphoreType.DMA,
        [pltpu.SemaphoreType.DMA] * NBUF,
        [pltpu.SemaphoreType.DMA] * NBUF,
        [pltpu.SemaphoreType.DMA] * NBUF,
    ],
)(_sc_body)


def _tc_body(x_ref, a0_ref, a1_ref, c_ref,
             w1_ref, w2_ref, b_ref, wp_ref, bp_ref, out_ref):
    a = a0_ref[...] + a1_ref[...]
    csum = jnp.sum(c_ref[...], axis=1)
    cnt = jnp.maximum(csum, 1.0)[:, None]
    mean = a / cnt
    h = jnp.dot(x_ref[...], w1_ref[...], preferred_element_type=jnp.float32)
    h = h + jnp.dot(mean, w2_ref[...], preferred_element_type=jnp.float32)
    h = jnp.maximum(h + b_ref[...], 0.0)
    p = jnp.dot(h, wp_ref[...], preferred_element_type=jnp.float32)
    out_ref[...] = jnp.maximum(p + bp_ref[...], 0.0)


def kernel(x, edge_index, W_sage, b_sage, W_proj, b_proj):
    ei = edge_index.astype(jnp.int32)
    npad = EPAD - N_EDGES
    src = jnp.concatenate([ei[0], jnp.zeros((npad,), jnp.int32)])
    dst = jnp.concatenate([ei[1], jnp.full((npad,), N_NODES, jnp.int32)])
    eidx = jnp.stack([src.reshape(-1, CH), dst.reshape(-1, CH)],
                     axis=1).reshape(-1, CH)

    zrow = jnp.zeros((ZCH, D), jnp.float32)
    iota = jnp.arange(NPAD, dtype=jnp.int32)

    acc, cnt = _sc_aggregate(eidx, x, zrow, iota)
    cnt = cnt.reshape(NW, NPAD).T

    R = 1000
    grid = (N_NODES // R,)
    row_spec = pl.BlockSpec((R, D), lambda i: (i, 0))
    cnt_spec = pl.BlockSpec((R, NW), lambda i: (i, 0))

    def w_spec(r, c_):
        return pl.BlockSpec((r, c_), lambda i: (0, 0))

    out = pl.pallas_call(
        _tc_body,
        grid=grid,
        in_specs=[
            row_spec, row_spec, row_spec, cnt_spec,
            w_spec(D, D), w_spec(D, D), w_spec(1, D), w_spec(D, 1), w_spec(1, 1),
        ],
        out_specs=pl.BlockSpec((R, 1), lambda i: (i, 0)),
        out_shape=jax.ShapeDtypeStruct((N_NODES, 1), jnp.float32),
    )(
        x, acc[:N_NODES], acc[NPAD:NPAD + N_NODES], cnt[:N_NODES],
        W_sage[:D], W_sage[D:], b_sage.reshape(1, D),
        W_proj, b_proj.reshape(1, 1),
    )
    return out
